# Initial kernel scaffold; baseline (speedup 1.0000x reference)
import jax
import jax.numpy as jnp
from jax.experimental import pallas as pl


def kernel(src_escrito_por, dst_escrito_por, src_escreveu, dst_escreveu, src_tem_genero, dst_tem_genero, src_pertence_a, dst_pertence_a, Wl1_e1, b1_e1, Wr1_e1, Wl1_e2, b1_e2, Wr1_e2, Wl1_e3, b1_e3, Wr1_e3, Wl1_e4, b1_e4, Wr1_e4, Wl2_e1, b2_e1, Wr2_e1, Wl2_e2, b2_e2, Wr2_e2, Wl2_e3, b2_e3, Wr2_e3, Wl2_e4, b2_e4, Wr2_e4):
    raise NotImplementedError("write your pallas kernel here")


# speedup vs baseline: 1.6645x; 1.6645x over previous
import functools

import jax
import jax.numpy as jnp
from jax import lax
from jax.experimental import pallas as pl
from jax.experimental.pallas import tpu as pltpu
from jax.experimental.pallas import tpu_sc as plsc

_N_LIVRO = 5000
_N_AUTOR = 2500
_N_GENERO = 100
_E = 10000
_HID = 256
_OUT = 128

_NC = 2
_NS = 16
_FS = _HID // _NS
_EPC = 6144
_EPAD = _NC * _EPC
_BATCH = 512
_NB = _EPC // _BATCH
_TL = 128
_NT = _BATCH // _TL

_NPAD = [2560, 5120, 128, 5120]
_NSRC = [_N_LIVRO, _N_AUTOR, _N_LIVRO, _N_GENERO]
_ACC_ROWS = 5120
_CROWS = 328


def _make_sc_segsum(with_counts):
    out_type = [jax.ShapeDtypeStruct((_NC, _NS, _NPAD[t], _FS), jnp.float32)
                for t in range(4)]
    if with_counts:
        out_type += [jax.ShapeDtypeStruct((_NC, _NPAD[t], _FS), jnp.float32)
                     for t in range(4)]
    scratch = [
        pltpu.VMEM((_EPC,), jnp.int32),
        pltpu.VMEM((_EPC,), jnp.int32),
        pltpu.VMEM((_EPC,), jnp.int32),
        pltpu.VMEM((_BATCH, _FS), jnp.float32),
        pltpu.VMEM((_ACC_ROWS, _FS), jnp.float32),
        pltpu.VMEM((_CROWS, _FS), jnp.float32),
        pltpu.SemaphoreType.DMA,
    ]
    mesh = plsc.VectorSubcoreMesh(core_axis_name="c", subcore_axis_name="s")

    @functools.partial(pl.kernel, mesh=mesh, out_type=out_type,
                       scratch_types=scratch,
                       compiler_params=pltpu.CompilerParams(
                           use_tc_tiling_on_sc=False))
    def k(tb0, tb1, tb2, tb3, sx0, sx1, sx2, sx3, dc0, dc1, dc2, dc3,
          zeros16, *rest):
        souts = rest[:4]
        couts = rest[4:8] if with_counts else None
        (sidx_v, gidx_v, didx_v, b_buf, acc, cacc, sem) = (
            rest[8 if with_counts else 4:])
        cid = lax.axis_index("c")
        sid = lax.axis_index("s")
        tables = [tb0, tb1, tb2, tb3]
        sxs = [sx0, sx1, sx2, sx3]
        dcs = [dc0, dc1, dc2, dc3]
        ones16 = jnp.full((_FS,), 1.0, jnp.float32)

        for t in range(4):
            npad = _NPAD[t]
            rp = npad // _NS
            lo = sid * rp
            pltpu.sync_copy(zeros16.at[pl.ds(0, npad)],
                            acc.at[pl.ds(0, npad)])
            if with_counts:
                pltpu.sync_copy(zeros16.at[pl.ds(0, _CROWS)], cacc)
            pltpu.sync_copy(sxs[t].at[cid], sidx_v)
            pltpu.sync_copy(dcs[t].at[cid], didx_v)
            off = sid * _NSRC[t]

            def adj(p, carry):
                sl = pl.ds(p * _FS, _FS)
                gidx_v[sl] = sidx_v[sl] + off
                return carry

            lax.fori_loop(0, _EPC // _FS, adj, 0)

            def batch(b, carry, rp=rp, lo=lo, t=t):
                gd = [pltpu.async_copy(
                        tables[t].at[gidx_v.at[pl.ds(b * _BATCH + q * _TL,
                                                     _TL)]],
                        b_buf.at[pl.ds(q * _TL, _TL)], sem)
                      for q in range(_NT)]
                for d in gd:
                    d.wait()

                def grp(g, c2):
                    dvec = didx_v[pl.ds(b * _BATCH + g * _FS, _FS)]
                    for lane in range(_FS):
                        d = lax.squeeze(
                            lax.slice(dvec, (lane,), (lane + 1,)), (0,))
                        gl = g * _FS + lane
                        acc[d, :] = acc[d, :] + b_buf[gl, :]
                        if with_counts:
                            ok = jnp.logical_and(d >= lo, d < lo + rp)
                            dci = jnp.where(ok, d - lo, _CROWS - 1)
                            cacc[dci, :] = cacc[dci, :] + ones16
                    return c2

                lax.fori_loop(0, _BATCH // _FS, grp, 0)
                return carry

            lax.fori_loop(0, _NB, batch, 0)

            pltpu.sync_copy(acc.at[pl.ds(0, npad)], souts[t].at[cid, sid])
            if with_counts:
                pltpu.sync_copy(cacc.at[pl.ds(0, rp)],
                                couts[t].at[cid, pl.ds(lo, rp)])

    return k


_sc_layer1 = _make_sc_segsum(with_counts=True)
_sc_layer2 = _make_sc_segsum(with_counts=False)


def _sum_from(s_ref):
    x = s_ref[...]
    s = x[0] + x[1]
    return jnp.transpose(s, (1, 0, 2)).reshape(s.shape[1], _HID)


def _cnt_from(c_ref):
    c = c_ref[...]
    return c[0, :, 0:1] + c[1, :, 0:1]


def _mean_from(s_ref, c_ref):
    return _sum_from(s_ref) / jnp.maximum(_cnt_from(c_ref), 1.0)


def _hT(h):
    return jnp.transpose(h.reshape(h.shape[0], _NS, _FS), (1, 0, 2))


def _h_one_body(s, c, b_ref, wr_ref, o_ref, ot_ref):
    mean = _mean_from(s, c)
    h = jnp.maximum(mean + b_ref[...] + wr_ref[...], 0.0)
    o_ref[...] = h
    ot_ref[...] = _hT(h)


def _h_two_body(s2, c2, b2_ref, wr2_ref, s4, c4, b4_ref, wr4_ref,
                o_ref, ot_ref):
    m2 = _mean_from(s2, c2) + b2_ref[...] + wr2_ref[...]
    m4 = _mean_from(s4, c4) + b4_ref[...] + wr4_ref[...]
    h = jnp.maximum((m2 + m4) * 0.5, 0.0)
    o_ref[...] = h
    ot_ref[...] = _hT(h)


def _l2norm(z):
    nrm = jnp.sqrt(jnp.sum(z * z, axis=1, keepdims=True))
    return z / jnp.maximum(nrm, 1e-12)


def _out_one_body(s, c, h_ref, wl_ref, b_ref, wr_ref, o_ref):
    m = _mean_from(s, c)
    z = (jnp.dot(m, wl_ref[...], preferred_element_type=jnp.float32)
         + b_ref[...]
         + jnp.dot(h_ref[...], wr_ref[...], preferred_element_type=jnp.float32))
    o_ref[...] = _l2norm(z)


def _out_two_body(s2, c2, wl2_ref, b2_ref, wr2_ref,
                  s4, c4, wl4_ref, b4_ref, wr4_ref, h_ref, o_ref):
    h = h_ref[...]
    z2 = (jnp.dot(_mean_from(s2, c2), wl2_ref[...],
                  preferred_element_type=jnp.float32)
          + b2_ref[...]
          + jnp.dot(h, wr2_ref[...], preferred_element_type=jnp.float32))
    z4 = (jnp.dot(_mean_from(s4, c4), wl4_ref[...],
                  preferred_element_type=jnp.float32)
          + b4_ref[...]
          + jnp.dot(h, wr4_ref[...], preferred_element_type=jnp.float32))
    o_ref[...] = _l2norm((z2 + z4) * 0.5)


def _spec_rows(bs, w):
    return pl.BlockSpec((bs, w), lambda i: (i, 0))


def _spec_full(shape):
    return pl.BlockSpec(shape, lambda i: (0, 0))


def _spec_S(bs):
    return pl.BlockSpec((_NC, _NS, bs, _FS), lambda i: (0, 0, i, 0))


def _spec_C(bs):
    return pl.BlockSpec((_NC, bs, _FS), lambda i: (0, i, 0))


def _spec_hT(bs):
    return pl.BlockSpec((_NS, bs, _FS), lambda i: (0, i, 0))


def _tc_h_one(S, C, b, Wr, n, bs):
    g = -(-n // bs)
    return pl.pallas_call(
        _h_one_body,
        grid=(g,),
        in_specs=[_spec_S(bs), _spec_C(bs), _spec_full((1, _HID)),
                  _spec_rows(bs, _HID)],
        out_specs=[_spec_rows(bs, _HID), _spec_hT(bs)],
        out_shape=[jax.ShapeDtypeStruct((n, _HID), jnp.float32),
                   jax.ShapeDtypeStruct((_NS, n, _FS), jnp.float32)],
    )(S, C, b.reshape(1, _HID), Wr)


def _tc_h_two(S2, C2, b2, Wr2, S4, C4, b4, Wr4, n, bs):
    g = -(-n // bs)
    return pl.pallas_call(
        _h_two_body,
        grid=(g,),
        in_specs=[_spec_S(bs), _spec_C(bs), _spec_full((1, _HID)),
                  _spec_rows(bs, _HID)] * 2,
        out_specs=[_spec_rows(bs, _HID), _spec_hT(bs)],
        out_shape=[jax.ShapeDtypeStruct((n, _HID), jnp.float32),
                   jax.ShapeDtypeStruct((_NS, n, _FS), jnp.float32)],
    )(S2, C2, b2.reshape(1, _HID), Wr2, S4, C4, b4.reshape(1, _HID), Wr4)


def _tc_out_one(S, C, h, Wl, b, Wr, n, bs):
    g = -(-n // bs)
    return pl.pallas_call(
        _out_one_body,
        grid=(g,),
        in_specs=[_spec_S(bs), _spec_C(bs), _spec_rows(bs, _HID),
                  _spec_full((_HID, _OUT)), _spec_full((1, _OUT)),
                  _spec_full((_HID, _OUT))],
        out_specs=_spec_rows(bs, _OUT),
        out_shape=jax.ShapeDtypeStruct((n, _OUT), jnp.float32),
    )(S, C, h, Wl, b.reshape(1, _OUT), Wr)


def _tc_out_two(S2, C2, Wl2, b2, Wr2, S4, C4, Wl4, b4, Wr4, h, n, bs):
    g = -(-n // bs)
    return pl.pallas_call(
        _out_two_body,
        grid=(g,),
        in_specs=[_spec_S(bs), _spec_C(bs), _spec_full((_HID, _OUT)),
                  _spec_full((1, _OUT)), _spec_full((_HID, _OUT))] * 2
                 + [_spec_rows(bs, _HID)],
        out_specs=_spec_rows(bs, _OUT),
        out_shape=jax.ShapeDtypeStruct((n, _OUT), jnp.float32),
    )(S2, C2, Wl2, b2.reshape(1, _OUT), Wr2,
      S4, C4, Wl4, b4.reshape(1, _OUT), Wr4, h)


def _slice_table(w):
    n = w.shape[0]
    return jnp.transpose(w.reshape(n, _NS, _FS), (1, 0, 2)).reshape(
        _NS * n, _FS)


def _prep_src(idx):
    pad = jnp.arange(_EPAD - _E, dtype=jnp.int32) % 64
    full = jnp.concatenate([idx.astype(jnp.int32), pad])
    return full.reshape(_NC, _EPC)


def _prep_dst(idx, n_dst):
    pad = jnp.full((_EPAD - _E,), n_dst, jnp.int32)
    full = jnp.concatenate([idx.astype(jnp.int32), pad])
    return full.reshape(_NC, _EPC)


def kernel(src_escrito_por, dst_escrito_por, src_escreveu, dst_escreveu,
           src_tem_genero, dst_tem_genero, src_pertence_a, dst_pertence_a,
           Wl1_e1, b1_e1, Wr1_e1, Wl1_e2, b1_e2, Wr1_e2,
           Wl1_e3, b1_e3, Wr1_e3, Wl1_e4, b1_e4, Wr1_e4,
           Wl2_e1, b2_e1, Wr2_e1, Wl2_e2, b2_e2, Wr2_e2,
           Wl2_e3, b2_e3, Wr2_e3, Wl2_e4, b2_e4, Wr2_e4):
    sx = [_prep_src(src_escrito_por), _prep_src(src_escreveu),
          _prep_src(src_tem_genero), _prep_src(src_pertence_a)]
    dc = [_prep_dst(dst_escrito_por, _N_AUTOR),
          _prep_dst(dst_escreveu, _N_LIVRO),
          _prep_dst(dst_tem_genero, _N_GENERO),
          _prep_dst(dst_pertence_a, _N_LIVRO)]
    zeros16 = jnp.zeros((_ACC_ROWS, _FS), jnp.float32)
    w1t = [_slice_table(Wl1_e1), _slice_table(Wl1_e2),
           _slice_table(Wl1_e3), _slice_table(Wl1_e4)]

    r1 = _sc_layer1(*w1t, *sx, *dc, zeros16)
    S1, C1 = r1[:4], r1[4:]

    h_autor, hT_autor = _tc_h_one(S1[0], C1[0], b1_e1, Wr1_e1, _N_AUTOR, 512)
    h_livro, hT_livro = _tc_h_two(S1[1], C1[1], b1_e2, Wr1_e2,
                                  S1[3], C1[3], b1_e4, Wr1_e4, _N_LIVRO, 512)
    h_genero, hT_genero = _tc_h_one(S1[2], C1[2], b1_e3, Wr1_e3,
                                    _N_GENERO, 128)

    h2t = [hT_livro.reshape(_NS * _N_LIVRO, _FS),
           hT_autor.reshape(_NS * _N_AUTOR, _FS),
           hT_livro.reshape(_NS * _N_LIVRO, _FS),
           hT_genero.reshape(_NS * _N_GENERO, _FS)]
    S2 = _sc_layer2(*h2t, *sx, *dc, zeros16)

    out_autor = _tc_out_one(S2[0], C1[0], h_autor, Wl2_e1, b2_e1, Wr2_e1,
                            _N_AUTOR, 512)
    out_livro = _tc_out_two(S2[1], C1[1], Wl2_e2, b2_e2, Wr2_e2,
                            S2[3], C1[3], Wl2_e4, b2_e4, Wr2_e4,
                            h_livro, _N_LIVRO, 512)
    out_genero = _tc_out_one(S2[2], C1[2], h_genero, Wl2_e3, b2_e3, Wr2_e3,
                             _N_GENERO, 128)
    return (out_livro, out_autor, out_genero)


# speedup vs baseline: 1.7201x; 1.0334x over previous
import functools

import jax
import jax.numpy as jnp
from jax import lax
from jax.experimental import pallas as pl
from jax.experimental.pallas import tpu as pltpu
from jax.experimental.pallas import tpu_sc as plsc

_N_LIVRO = 5000
_N_AUTOR = 2500
_N_GENERO = 100
_E = 10000
_HID = 256
_OUT = 128

_NC = 2
_NS = 16
_FS = _HID // _NS
_EPC = 6144
_EPAD = _NC * _EPC
_BATCH = 512
_NB = _EPC // _BATCH
_TL = 128
_NT = _BATCH // _TL

_NPAD = [2560, 5120, 128, 5120]
_NSRC = [_N_LIVRO, _N_AUTOR, _N_LIVRO, _N_GENERO]
_ACC_ROWS = 5120
_CROWS = 328


def _make_sc_segsum(with_counts):
    out_type = [jax.ShapeDtypeStruct((_NC, _NS, _NPAD[t], _FS), jnp.float32)
                for t in range(4)]
    if with_counts:
        out_type += [jax.ShapeDtypeStruct((_NC, _NPAD[t], _FS), jnp.float32)
                     for t in range(4)]
    scratch = [
        pltpu.VMEM((_EPC,), jnp.int32),
        pltpu.VMEM((_EPC,), jnp.int32),
        pltpu.VMEM((_EPC,), jnp.int32),
        pltpu.VMEM((_BATCH, _FS), jnp.float32),
        pltpu.VMEM((_BATCH, _FS), jnp.float32),
        pltpu.VMEM((_ACC_ROWS, _FS), jnp.float32),
        pltpu.VMEM((_CROWS, _FS), jnp.float32),
        pltpu.SemaphoreType.DMA,
        pltpu.SemaphoreType.DMA,
    ]
    mesh = plsc.VectorSubcoreMesh(core_axis_name="c", subcore_axis_name="s")

    @functools.partial(pl.kernel, mesh=mesh, out_type=out_type,
                       scratch_types=scratch,
                       compiler_params=pltpu.CompilerParams(
                           use_tc_tiling_on_sc=False))
    def k(tb0, tb1, tb2, tb3, sx0, sx1, sx2, sx3, dc0, dc1, dc2, dc3,
          zeros16, *rest):
        souts = rest[:4]
        couts = rest[4:8] if with_counts else None
        (sidx_v, gidx_v, didx_v, b_buf0, b_buf1, acc, cacc, sem0, sem1) = (
            rest[8 if with_counts else 4:])
        cid = lax.axis_index("c")
        sid = lax.axis_index("s")
        tables = [tb0, tb1, tb2, tb3]
        sxs = [sx0, sx1, sx2, sx3]
        dcs = [dc0, dc1, dc2, dc3]
        ones16 = jnp.full((_FS,), 1.0, jnp.float32)

        for t in range(4):
            npad = _NPAD[t]
            rp = npad // _NS
            lo = sid * rp
            pltpu.sync_copy(zeros16.at[pl.ds(0, npad)],
                            acc.at[pl.ds(0, npad)])
            if with_counts:
                pltpu.sync_copy(zeros16.at[pl.ds(0, _CROWS)], cacc)
            pltpu.sync_copy(sxs[t].at[cid], sidx_v)
            pltpu.sync_copy(dcs[t].at[cid], didx_v)
            off = sid * _NSRC[t]

            def adj(p, carry):
                sl = pl.ds(p * _FS, _FS)
                gidx_v[sl] = sidx_v[sl] + off
                return carry

            lax.fori_loop(0, _EPC // _FS, adj, 0)

            def issue(b, buf, sem, t=t):
                return [pltpu.async_copy(
                            tables[t].at[gidx_v.at[pl.ds(b * _BATCH + q * _TL,
                                                         _TL)]],
                            buf.at[pl.ds(q * _TL, _TL)], sem)
                        for q in range(_NT)]

            def rmw(b, buf, rp=rp, lo=lo):
                def grp(g, c2):
                    dvec = didx_v[pl.ds(b * _BATCH + g * _FS, _FS)]
                    for lane in range(_FS):
                        d = lax.squeeze(
                            lax.slice(dvec, (lane,), (lane + 1,)), (0,))
                        gl = g * _FS + lane
                        acc[d, :] = acc[d, :] + buf[gl, :]
                        if with_counts:
                            ok = jnp.logical_and(d >= lo, d < lo + rp)
                            dci = jnp.where(ok, d - lo, _CROWS - 1)
                            cacc[dci, :] = cacc[dci, :] + ones16
                    return c2

                lax.fori_loop(0, _BATCH // _FS, grp, 0)

            def batch2(bb, carry):
                b0 = bb * 2
                gd0 = issue(b0, b_buf0, sem0)
                gd1 = issue(b0 + 1, b_buf1, sem1)
                for d in gd0:
                    d.wait()
                rmw(b0, b_buf0)
                for d in gd1:
                    d.wait()
                rmw(b0 + 1, b_buf1)
                return carry

            lax.fori_loop(0, _NB // 2, batch2, 0)

            pltpu.sync_copy(acc.at[pl.ds(0, npad)], souts[t].at[cid, sid])
            if with_counts:
                pltpu.sync_copy(cacc.at[pl.ds(0, rp)],
                                couts[t].at[cid, pl.ds(lo, rp)])

    return k


_sc_layer1 = _make_sc_segsum(with_counts=True)
_sc_layer2 = _make_sc_segsum(with_counts=False)


def _sum_from(s_ref):
    x = s_ref[...]
    s = x[0] + x[1]
    return jnp.transpose(s, (1, 0, 2)).reshape(s.shape[1], _HID)


def _cnt_from(c_ref):
    c = c_ref[...]
    return c[0, :, 0:1] + c[1, :, 0:1]


def _mean_from(s_ref, c_ref):
    return _sum_from(s_ref) / jnp.maximum(_cnt_from(c_ref), 1.0)


def _hT(h):
    return jnp.transpose(h.reshape(h.shape[0], _NS, _FS), (1, 0, 2))


def _h_one_body(s, c, b_ref, wr_ref, o_ref, ot_ref):
    mean = _mean_from(s, c)
    h = jnp.maximum(mean + b_ref[...] + wr_ref[...], 0.0)
    o_ref[...] = h
    ot_ref[...] = _hT(h)


def _h_two_body(s2, c2, b2_ref, wr2_ref, s4, c4, b4_ref, wr4_ref,
                o_ref, ot_ref):
    m2 = _mean_from(s2, c2) + b2_ref[...] + wr2_ref[...]
    m4 = _mean_from(s4, c4) + b4_ref[...] + wr4_ref[...]
    h = jnp.maximum((m2 + m4) * 0.5, 0.0)
    o_ref[...] = h
    ot_ref[...] = _hT(h)


def _l2norm(z):
    nrm = jnp.sqrt(jnp.sum(z * z, axis=1, keepdims=True))
    return z / jnp.maximum(nrm, 1e-12)


def _out_one_body(s, c, h_ref, wl_ref, b_ref, wr_ref, o_ref):
    m = _mean_from(s, c)
    z = (jnp.dot(m, wl_ref[...], preferred_element_type=jnp.float32)
         + b_ref[...]
         + jnp.dot(h_ref[...], wr_ref[...], preferred_element_type=jnp.float32))
    o_ref[...] = _l2norm(z)


def _out_two_body(s2, c2, wl2_ref, b2_ref, wr2_ref,
                  s4, c4, wl4_ref, b4_ref, wr4_ref, h_ref, o_ref):
    h = h_ref[...]
    z2 = (jnp.dot(_mean_from(s2, c2), wl2_ref[...],
                  preferred_element_type=jnp.float32)
          + b2_ref[...]
          + jnp.dot(h, wr2_ref[...], preferred_element_type=jnp.float32))
    z4 = (jnp.dot(_mean_from(s4, c4), wl4_ref[...],
                  preferred_element_type=jnp.float32)
          + b4_ref[...]
          + jnp.dot(h, wr4_ref[...], preferred_element_type=jnp.float32))
    o_ref[...] = _l2norm((z2 + z4) * 0.5)


def _spec_rows(bs, w):
    return pl.BlockSpec((bs, w), lambda i: (i, 0))


def _spec_full(shape):
    return pl.BlockSpec(shape, lambda i: (0, 0))


def _spec_S(bs):
    return pl.BlockSpec((_NC, _NS, bs, _FS), lambda i: (0, 0, i, 0))


def _spec_C(bs):
    return pl.BlockSpec((_NC, bs, _FS), lambda i: (0, i, 0))


def _spec_hT(bs):
    return pl.BlockSpec((_NS, bs, _FS), lambda i: (0, i, 0))


def _tc_h_one(S, C, b, Wr, n, bs):
    g = -(-n // bs)
    return pl.pallas_call(
        _h_one_body,
        grid=(g,),
        in_specs=[_spec_S(bs), _spec_C(bs), _spec_full((1, _HID)),
                  _spec_rows(bs, _HID)],
        out_specs=[_spec_rows(bs, _HID), _spec_hT(bs)],
        out_shape=[jax.ShapeDtypeStruct((n, _HID), jnp.float32),
                   jax.ShapeDtypeStruct((_NS, n, _FS), jnp.float32)],
    )(S, C, b.reshape(1, _HID), Wr)


def _tc_h_two(S2, C2, b2, Wr2, S4, C4, b4, Wr4, n, bs):
    g = -(-n // bs)
    return pl.pallas_call(
        _h_two_body,
        grid=(g,),
        in_specs=[_spec_S(bs), _spec_C(bs), _spec_full((1, _HID)),
                  _spec_rows(bs, _HID)] * 2,
        out_specs=[_spec_rows(bs, _HID), _spec_hT(bs)],
        out_shape=[jax.ShapeDtypeStruct((n, _HID), jnp.float32),
                   jax.ShapeDtypeStruct((_NS, n, _FS), jnp.float32)],
    )(S2, C2, b2.reshape(1, _HID), Wr2, S4, C4, b4.reshape(1, _HID), Wr4)


def _tc_out_one(S, C, h, Wl, b, Wr, n, bs):
    g = -(-n // bs)
    return pl.pallas_call(
        _out_one_body,
        grid=(g,),
        in_specs=[_spec_S(bs), _spec_C(bs), _spec_rows(bs, _HID),
                  _spec_full((_HID, _OUT)), _spec_full((1, _OUT)),
                  _spec_full((_HID, _OUT))],
        out_specs=_spec_rows(bs, _OUT),
        out_shape=jax.ShapeDtypeStruct((n, _OUT), jnp.float32),
    )(S, C, h, Wl, b.reshape(1, _OUT), Wr)


def _tc_out_two(S2, C2, Wl2, b2, Wr2, S4, C4, Wl4, b4, Wr4, h, n, bs):
    g = -(-n // bs)
    return pl.pallas_call(
        _out_two_body,
        grid=(g,),
        in_specs=[_spec_S(bs), _spec_C(bs), _spec_full((_HID, _OUT)),
                  _spec_full((1, _OUT)), _spec_full((_HID, _OUT))] * 2
                 + [_spec_rows(bs, _HID)],
        out_specs=_spec_rows(bs, _OUT),
        out_shape=jax.ShapeDtypeStruct((n, _OUT), jnp.float32),
    )(S2, C2, Wl2, b2.reshape(1, _OUT), Wr2,
      S4, C4, Wl4, b4.reshape(1, _OUT), Wr4, h)


def _slice_table(w):
    n = w.shape[0]
    return jnp.transpose(w.reshape(n, _NS, _FS), (1, 0, 2)).reshape(
        _NS * n, _FS)


def _prep_src(idx):
    pad = jnp.arange(_EPAD - _E, dtype=jnp.int32) % 64
    full = jnp.concatenate([idx.astype(jnp.int32), pad])
    return full.reshape(_NC, _EPC)


def _prep_dst(idx, n_dst):
    pad = jnp.full((_EPAD - _E,), n_dst, jnp.int32)
    full = jnp.concatenate([idx.astype(jnp.int32), pad])
    return full.reshape(_NC, _EPC)


def kernel(src_escrito_por, dst_escrito_por, src_escreveu, dst_escreveu,
           src_tem_genero, dst_tem_genero, src_pertence_a, dst_pertence_a,
           Wl1_e1, b1_e1, Wr1_e1, Wl1_e2, b1_e2, Wr1_e2,
           Wl1_e3, b1_e3, Wr1_e3, Wl1_e4, b1_e4, Wr1_e4,
           Wl2_e1, b2_e1, Wr2_e1, Wl2_e2, b2_e2, Wr2_e2,
           Wl2_e3, b2_e3, Wr2_e3, Wl2_e4, b2_e4, Wr2_e4):
    sx = [_prep_src(src_escrito_por), _prep_src(src_escreveu),
          _prep_src(src_tem_genero), _prep_src(src_pertence_a)]
    dc = [_prep_dst(dst_escrito_por, _N_AUTOR),
          _prep_dst(dst_escreveu, _N_LIVRO),
          _prep_dst(dst_tem_genero, _N_GENERO),
          _prep_dst(dst_pertence_a, _N_LIVRO)]
    zeros16 = jnp.zeros((_ACC_ROWS, _FS), jnp.float32)
    w1t = [_slice_table(Wl1_e1), _slice_table(Wl1_e2),
           _slice_table(Wl1_e3), _slice_table(Wl1_e4)]

    r1 = _sc_layer1(*w1t, *sx, *dc, zeros16)
    S1, C1 = r1[:4], r1[4:]

    h_autor, hT_autor = _tc_h_one(S1[0], C1[0], b1_e1, Wr1_e1, _N_AUTOR, 512)
    h_livro, hT_livro = _tc_h_two(S1[1], C1[1], b1_e2, Wr1_e2,
                                  S1[3], C1[3], b1_e4, Wr1_e4, _N_LIVRO, 512)
    h_genero, hT_genero = _tc_h_one(S1[2], C1[2], b1_e3, Wr1_e3,
                                    _N_GENERO, 128)

    h2t = [hT_livro.reshape(_NS * _N_LIVRO, _FS),
           hT_autor.reshape(_NS * _N_AUTOR, _FS),
           hT_livro.reshape(_NS * _N_LIVRO, _FS),
           hT_genero.reshape(_NS * _N_GENERO, _FS)]
    S2 = _sc_layer2(*h2t, *sx, *dc, zeros16)

    out_autor = _tc_out_one(S2[0], C1[0], h_autor, Wl2_e1, b2_e1, Wr2_e1,
                            _N_AUTOR, 512)
    out_livro = _tc_out_two(S2[1], C1[1], Wl2_e2, b2_e2, Wr2_e2,
                            S2[3], C1[3], Wl2_e4, b2_e4, Wr2_e4,
                            h_livro, _N_LIVRO, 512)
    out_genero = _tc_out_one(S2[2], C1[2], h_genero, Wl2_e3, b2_e3, Wr2_e3,
                             _N_GENERO, 128)
    return (out_livro, out_autor, out_genero)
